# baseline (speedup 1.0000x reference)
import functools

import jax
import jax.numpy as jnp
from jax import lax
from jax.experimental import pallas as pl
from jax.experimental.pallas import tpu as pltpu
from jax.experimental.pallas import tpu_sc as plsc

B, S, M, E, F = 2, 4096, 1024, 64, 1024
T = B * S
CAP = 128
NSLOT = (E + 1) * CAP
ZROW = E * CAP
TRASH = NSLOT - 1
CHUNK = 512
NCHUNK = T // CHUNK
GW = 128

NC, NS = 2, 16
NW = NC * NS
TPW = T // NW
CH = 32
NCH = TPW // CH
NBUF = 3
KLAG = 1
PAD_PER_TILE = (NSLOT - E * CAP) // NW


def _router_body(x_ref, wg_ref, dest_ref, src_ref, gate_ref, xbf_ref,
                 laux_ref, base_ref, sumg_ref, tri_ref):
    i = pl.program_id(0)

    @pl.when(i == 0)
    def _init():
        base_ref[...] = jnp.zeros_like(base_ref)
        sumg_ref[...] = jnp.zeros_like(sumg_ref)
        r = lax.broadcasted_iota(jnp.int32, (CHUNK, CHUNK), 0)
        c = lax.broadcasted_iota(jnp.int32, (CHUNK, CHUNK), 1)
        tri_ref[...] = (r >= c).astype(jnp.float32)

    x = x_ref[...]
    logits = jnp.dot(x, wg_ref[...], preferred_element_type=jnp.float32)
    rowmax = jnp.max(logits, axis=1, keepdims=True)
    ez = jnp.exp(logits - rowmax)
    gates = ez / jnp.sum(ez, axis=1, keepdims=True)

    lane = lax.broadcasted_iota(jnp.int32, (CHUNK, E), 1)
    ismax = logits == rowmax
    aidx = jnp.min(jnp.where(ismax, lane, E), axis=1, keepdims=True)
    onehot = (lane == aidx).astype(jnp.float32)

    csum = jnp.dot(tri_ref[...], onehot, preferred_element_type=jnp.float32)
    locations = csum - 1.0 + base_ref[...]
    keep = jnp.where(locations < CAP, onehot, 0.0)
    locf = jnp.sum(locations * keep, axis=1)
    gate_s = jnp.sum(gates * keep, axis=1)
    validb = jnp.sum(keep, axis=1) > 0.0

    slot = aidx[:, 0] * CAP + locf.astype(jnp.int32)
    dest_ref[...] = jnp.where(validb, slot, TRASH).reshape(1, 1, CHUNK)
    src_ref[...] = jnp.where(validb, slot, ZROW).reshape(1, 1, CHUNK)
    gate_ref[...] = jnp.broadcast_to(gate_s[:, None], (CHUNK, GW))
    u = lax.bitcast_convert_type(x, jnp.int32)
    rnd = u + 0x7FFF + jnp.bitwise_and(lax.shift_right_logical(u, 16), 1)
    hi = jnp.bitwise_and(rnd[:, :M // 2], jnp.int32(-65536))
    lo = lax.shift_right_logical(rnd[:, M // 2:], 16)
    xbf_ref[...] = jnp.bitwise_or(hi, lo)

    sumg_ref[...] += jnp.sum(gates, axis=0, keepdims=True)
    base_ref[...] += csum[CHUNK - 1:CHUNK, :]

    @pl.when(i == NCHUNK - 1)
    def _fin():
        laux_ref[0, 0] = jnp.sum(sumg_ref[...] * base_ref[...]) * (E / (T * T))


def _router(x, wg):
    return pl.pallas_call(
        _router_body,
        grid=(NCHUNK,),
        in_specs=[
            pl.BlockSpec((CHUNK, M), lambda i: (i, 0)),
            pl.BlockSpec((M, E), lambda i: (0, 0)),
        ],
        out_specs=[
            pl.BlockSpec((1, 1, CHUNK), lambda i: (i, 0, 0)),
            pl.BlockSpec((1, 1, CHUNK), lambda i: (i, 0, 0)),
            pl.BlockSpec((CHUNK, GW), lambda i: (i, 0)),
            pl.BlockSpec((CHUNK, M // 2), lambda i: (i, 0)),
            pl.BlockSpec(memory_space=pltpu.SMEM, block_shape=(1, 1),
                         index_map=lambda i: (0, 0)),
        ],
        out_shape=[
            jax.ShapeDtypeStruct((NCHUNK, 1, CHUNK), jnp.int32),
            jax.ShapeDtypeStruct((NCHUNK, 1, CHUNK), jnp.int32),
            jax.ShapeDtypeStruct((T, GW), jnp.float32),
            jax.ShapeDtypeStruct((T, M // 2), jnp.int32),
            jax.ShapeDtypeStruct((1, 1), jnp.float32),
        ],
        scratch_shapes=[
            pltpu.VMEM((1, E), jnp.float32),
            pltpu.VMEM((1, E), jnp.float32),
            pltpu.VMEM((CHUNK, CHUNK), jnp.float32),
        ],
    )(x, wg)


CHD = 64
NCHD = TPW // CHD
NBUFD = 2
KLAGD = 1


def _dispatch(x, gate16, dest):
    mesh = plsc.VectorSubcoreMesh(core_axis_name="c", subcore_axis_name="s")

    @functools.partial(
        pl.kernel,
        out_type=[
            jax.ShapeDtypeStruct((NSLOT, M // 2), jnp.int32),
            jax.ShapeDtypeStruct((NSLOT, GW), jnp.float32),
        ],
        mesh=mesh,
        scratch_types=[
            pltpu.VMEM((NCHD, CHD), jnp.int32),
            [pltpu.VMEM((CHD, M // 2), jnp.int32)] * NBUFD,
            [pltpu.VMEM((CHD, GW), jnp.float32)] * NBUFD,
            [pltpu.SemaphoreType.DMA] * NBUFD,
            [pltpu.SemaphoreType.DMA] * NBUFD,
            [pltpu.SemaphoreType.DMA] * NBUFD,
            [pltpu.SemaphoreType.DMA] * NBUFD,
            pltpu.SemaphoreType.DMA,
        ],
    )
    def k(x_hbm, g_hbm, dest_hbm, disp_out, gslot_out,
          idx_v, rows, gbufs, semi, semgi, semo, semgo, semz):
        wid = lax.axis_index("s") * NC + lax.axis_index("c")
        tok0 = wid * TPW

        pltpu.sync_copy(dest_hbm.at[pl.ds(wid * NCHD, NCHD)], idx_v)

        for r in range(PAD_PER_TILE):
            def zcol(c, __, r=r):
                rows[0][r, pl.ds(c * 16, 16)] = jnp.zeros((16,), jnp.int32)
                return __
            lax.fori_loop(0, M // 32, zcol, 0)
            def zgcol(c, __, r=r):
                gbufs[0][r, pl.ds(c * 16, 16)] = jnp.zeros((16,), jnp.float32)
                return __
            lax.fori_loop(0, GW // 16, zgcol, 0)
        pad0 = E * CAP + wid * PAD_PER_TILE
        zc1 = pltpu.async_copy(rows[0].at[pl.ds(0, PAD_PER_TILE)],
                               disp_out.at[pl.ds(pad0, PAD_PER_TILE)], semz)
        zc2 = pltpu.async_copy(gbufs[0].at[pl.ds(0, PAD_PER_TILE)],
                               gslot_out.at[pl.ds(pad0, PAD_PER_TILE)], semz)

        incps = [None] * NCHD
        outs = [None] * NCHD

        def start_in(b):
            s = b % NBUFD
            incps[b] = (
                pltpu.async_copy(x_hbm.at[pl.ds(tok0 + b * CHD, CHD)],
                                 rows[s], semi[s]),
                pltpu.async_copy(g_hbm.at[pl.ds(tok0 + b * CHD, CHD)],
                                 gbufs[s], semgi[s]),
            )

        zc1.wait()
        zc2.wait()
        for b in range(min(NBUFD, NCHD)):
            start_in(b)
        drained = 0
        for b in range(NCHD):
            s = b % NBUFD
            for cp in incps[b]:
                cp.wait()
            outs[b] = (
                pltpu.async_copy(rows[s], disp_out.at[idx_v.at[b]], semo[s]),
                pltpu.async_copy(gbufs[s], gslot_out.at[idx_v.at[b]],
                                 semgo[s]),
            )
            j = b - KLAGD
            if j >= 0 and j + NBUFD < NCHD:
                for cp in outs[j]:
                    cp.wait()
                drained = j + 1
                start_in(j + NBUFD)
        for j in range(drained, NCHD):
            for cp in outs[j]:
                cp.wait()

    return k(x, gate16, dest.reshape(T // CHD, CHD))


def _ffn_body(disp_ref, g_ref, w1_ref, b1_ref, w2_ref, b2_ref, out_ref):
    w = disp_ref[...]
    x1 = lax.bitcast_convert_type(
        jnp.bitwise_and(w, jnp.int32(-65536)), jnp.float32)
    x2 = lax.bitcast_convert_type(lax.shift_left(w, 16), jnp.float32)
    xb = jnp.concatenate([x1, x2], axis=1)
    h = jnp.dot(xb, w1_ref[0], preferred_element_type=jnp.float32)
    h = jnp.maximum(h + b1_ref[0], 0.0)
    o = jnp.dot(h, w2_ref[0], preferred_element_type=jnp.float32)
    out_ref[...] = (o + b2_ref[0]) * g_ref[:, 0:1]


def _ffn(disp, gslot, W1, b1, W2, b2):
    ew = lambda e: (jnp.minimum(e, E - 1), 0, 0)
    return pl.pallas_call(
        _ffn_body,
        grid=(NSLOT // CAP,),
        in_specs=[
            pl.BlockSpec((CAP, M // 2), lambda e: (e, 0)),
            pl.BlockSpec((CAP, GW), lambda e: (e, 0)),
            pl.BlockSpec((1, M, F), ew),
            pl.BlockSpec((1, 1, F), ew),
            pl.BlockSpec((1, F, M), ew),
            pl.BlockSpec((1, 1, M), ew),
        ],
        out_specs=pl.BlockSpec((CAP, M), lambda e: (e, 0)),
        out_shape=jax.ShapeDtypeStruct((NSLOT, M), jnp.float32),
    )(disp, gslot, W1, b1.reshape(E, 1, F), W2, b2.reshape(E, 1, M))


def _combine(eo, src):
    mesh = plsc.VectorSubcoreMesh(core_axis_name="c", subcore_axis_name="s")

    @functools.partial(
        pl.kernel,
        out_type=jax.ShapeDtypeStruct((T, M), jnp.float32),
        mesh=mesh,
        scratch_types=[
            pltpu.VMEM((NCH, CH), jnp.int32),
            [pltpu.VMEM((CH, M), jnp.float32)] * NBUF,
            [pltpu.SemaphoreType.DMA] * NBUF,
            [pltpu.SemaphoreType.DMA] * NBUF,
        ],
    )
    def k(eo_hbm, src_hbm, y_out, idx_v, rows, semi, semo):
        wid = lax.axis_index("s") * NC + lax.axis_index("c")
        tok0 = wid * TPW
        pltpu.sync_copy(src_hbm.at[pl.ds(wid * NCH, NCH)], idx_v)

        incps = [None] * NCH
        outs = [None] * NCH

        def start_in(b):
            s = b % NBUF
            incps[b] = pltpu.async_copy(eo_hbm.at[idx_v.at[b]], rows[s],
                                        semi[s])

        for b in range(min(NBUF, NCH)):
            start_in(b)
        drained = 0
        for b in range(NCH):
            s = b % NBUF
            incps[b].wait()
            outs[b] = pltpu.async_copy(
                rows[s], y_out.at[pl.ds(tok0 + b * CH, CH)], semo[s])
            j = b - KLAG
            if j >= 0 and j + NBUF < NCH:
                outs[j].wait()
                drained = j + 1
                start_in(j + NBUF)
        for j in range(drained, NCH):
            outs[j].wait()

    return k(eo, src.reshape(T // CH, CH))


def kernel(hidden_states, wg, W1, b1, W2, b2):
    x = hidden_states.reshape(T, M)
    dest, src, gate16, xbf, laux = _router(x, wg)
    dest = dest.reshape(T)
    src = src.reshape(T)
    disp, gslot = _dispatch(xbf, gate16, dest)
    eo = _ffn(disp, gslot, W1, b1, W2, b2)
    y = _combine(eo, src)
    return (y.reshape(B, S, M), laux[0, 0])
